# Initial kernel scaffold; baseline (speedup 1.0000x reference)
import jax
import jax.numpy as jnp
from jax.experimental import pallas as pl


def kernel(x, edge, W1, b1, g1, bt1, W2, b2, g2, bt2, W3, b3):
    raise NotImplementedError("write your pallas kernel here")


# speedup vs baseline: 5.7693x; 5.7693x over previous
import functools

import jax
import jax.numpy as jnp
from jax import lax
from jax.experimental import pallas as pl
from jax.experimental.pallas import tpu as pltpu
from jax.experimental.pallas import tpu_sc as plsc

N = 10000
E = 320000
F = 64
EPS = 1e-5

NC, NS, WLEN = 2, 16, 128
NWIN = 160
EPAD = NS * NWIN * WLEN
NPAD = 10112
RP = NPAD // NS

_MESH = plsc.VectorSubcoreMesh(
    core_axis_name="c", subcore_axis_name="s", num_cores=NC, num_subcores=NS)


@functools.partial(
    pl.kernel,
    out_type=[jax.ShapeDtypeStruct((NPAD, F), jnp.float32)] * 2,
    mesh=_MESH,
    compiler_params=pltpu.CompilerParams(use_tc_tiling_on_sc=False),
    scratch_types=[
        pltpu.VMEM((NWIN, WLEN), jnp.int32),
        pltpu.VMEM((NWIN, WLEN), jnp.int32),
        pltpu.VMEM((WLEN, F), jnp.float32),
        pltpu.VMEM_SHARED((NPAD, F), jnp.float32),
        pltpu.SemaphoreType.DMA,
    ],
)
def _hop(tab0, tab1, zrows, sidx_hbm, didx_hbm, out0, out1,
         sidx, didx, rows, acc, sem):
    c = lax.axis_index("c")
    s = lax.axis_index("s")
    pltpu.sync_copy(sidx_hbm.at[s], sidx)
    pltpu.sync_copy(didx_hbm.at[s], didx)
    pltpu.sync_copy(zrows, acc.at[pl.ds(s * RP, RP)])
    plsc.subcore_barrier()

    def run(tab, out):
        def win(j, carry):
            pltpu.async_copy(tab.at[sidx.at[j]], rows, sem).wait()
            pltpu.sync_copy(rows, acc.at[didx.at[j]], add=True)
            return carry
        lax.fori_loop(0, NWIN, win, 0)
        plsc.subcore_barrier()
        pltpu.sync_copy(acc.at[pl.ds(s * RP, RP)], out.at[pl.ds(s * RP, RP)])

    @pl.when(c == 0)
    def _():
        run(tab0, out0)

    @pl.when(c == 1)
    def _():
        run(tab1, out1)


@functools.partial(
    pl.kernel,
    out_type=[jax.ShapeDtypeStruct((NPAD, 16), jnp.float32)] * 2,
    mesh=_MESH,
    compiler_params=pltpu.CompilerParams(use_tc_tiling_on_sc=False),
    scratch_types=[
        pltpu.VMEM((NWIN, WLEN), jnp.int32),
        pltpu.VMEM((WLEN, 16), jnp.float32),
        pltpu.VMEM_SHARED((NPAD, 16), jnp.float32),
    ],
)
def _counts(ones_hbm, z16, nidx_hbm, hidx_hbm, dcnt, bcnt, idx, ones, acc):
    c = lax.axis_index("c")
    s = lax.axis_index("s")
    pltpu.sync_copy(ones_hbm, ones)
    pltpu.sync_copy(z16, acc.at[pl.ds(s * RP, RP)])

    def run(idx_hbm, out):
        pltpu.sync_copy(idx_hbm.at[s], idx)
        plsc.subcore_barrier()

        def win(j, carry):
            pltpu.sync_copy(ones, acc.at[idx.at[j]], add=True)
            return carry
        lax.fori_loop(0, NWIN, win, 0)
        plsc.subcore_barrier()
        pltpu.sync_copy(acc.at[pl.ds(s * RP, RP)], out.at[pl.ds(s * RP, RP)])

    @pl.when(c == 0)
    def _():
        run(nidx_hbm, dcnt)

    @pl.when(c == 1)
    def _():
        run(hidx_hbm, bcnt)


def _mm1_body(x_ref, w_ref, *o_refs):
    xw = jnp.dot(x_ref[...], w_ref[...], preferred_element_type=jnp.float32)
    for q, o in enumerate(o_refs):
        o[...] = xw[:, q * F:(q + 1) * F]


_mm1 = pl.pallas_call(
    _mm1_body,
    out_shape=[jax.ShapeDtypeStruct((NPAD, F), jnp.float32)] * 4,
)


def _scale_body(a0, a1, a2, a3, cnt_ref, o0, o1, o2, o3):
    cnt = cnt_ref[...][:, :1]
    inv = jnp.where(cnt > 0, 1.0 / cnt, 0.0)
    for a, o in ((a0, o0), (a1, o1), (a2, o2), (a3, o3)):
        o[...] = a[...] * inv


_scale = pl.pallas_call(
    _scale_body,
    out_shape=[jax.ShapeDtypeStruct((NPAD, F), jnp.float32)] * 4,
)


def _fin_body(a0, a1, a2, a3, cnt_ref, b_ref, g_ref, bt_ref, w_ref, *o_refs):
    t = jnp.concatenate([a0[...], a1[...], a2[...], a3[...]], axis=1)
    cnt = cnt_ref[...][:, :1]
    inv = jnp.where(cnt > 0, 1.0 / cnt, 0.0)
    t = jnp.maximum(t * inv + b_ref[...], 0.0)[:N]
    m = jnp.mean(t, axis=0, keepdims=True)
    v = jnp.mean(jnp.square(t - m), axis=0, keepdims=True)
    t = (t - m) * lax.rsqrt(v + EPS) * g_ref[...] + bt_ref[...]
    xw = jnp.dot(t, w_ref[...], preferred_element_type=jnp.float32)
    xw = jnp.concatenate([xw, jnp.zeros((NPAD - N, 4 * F), xw.dtype)], axis=0)
    for q, o in enumerate(o_refs):
        o[...] = xw[:, q * F:(q + 1) * F]


_fin = pl.pallas_call(
    _fin_body,
    out_shape=[jax.ShapeDtypeStruct((NPAD, F), jnp.float32)] * 4,
)


def _fin3_body(a0, a1, a2, a3, cnt_ref, b_ref, o_ref):
    t = jnp.concatenate([a0[...], a1[...], a2[...], a3[...]], axis=1)
    cnt = cnt_ref[...][:, :1]
    inv = jnp.where(cnt > 0, 1.0 / cnt, 0.0)
    o_ref[...] = jnp.maximum(t * inv + b_ref[...], 0.0)[:N]


_fin3 = pl.pallas_call(
    _fin3_body,
    out_shape=jax.ShapeDtypeStruct((N, 4 * F), jnp.float32),
)


def _tile_windows(idx):
    pad = jnp.arange(EPAD - E, dtype=jnp.int32) % (NPAD - N) + N
    return jnp.concatenate([idx.astype(jnp.int32), pad]).reshape(NS, NWIN, WLEN)


def _seg_hop(q, zrows, sidx, didx):
    a0, a1 = _hop(q[0], q[1], zrows, sidx, didx)
    a2, a3 = _hop(q[2], q[3], zrows, sidx, didx)
    return (a0, a1, a2, a3)


def kernel(x, edge, W1, b1, g1, bt1, W2, b2, g2, bt2, W3, b3):
    nwin = _tile_windows(edge[0])
    hwin = _tile_windows(edge[1])
    zrows = jnp.zeros((RP, F), jnp.float32)
    z16 = jnp.zeros((RP, 16), jnp.float32)
    ones = jnp.ones((WLEN, 16), jnp.float32)

    dcnt, bcnt = _counts(ones, z16, nwin, hwin)
    xp = jnp.pad(x, ((0, NPAD - N), (0, 0)))
    q = _mm1(xp, W1)

    for b, g, bt, wn in ((b1, g1, bt1, W2), (b2, g2, bt2, W3)):
        ae = _seg_hop(q, zrows, nwin, hwin)
        sq = _scale(*ae, bcnt)
        an = _seg_hop(sq, zrows, hwin, nwin)
        q = _fin(*an, dcnt, b[None], g[None], bt[None], wn)

    ae = _seg_hop(q, zrows, nwin, hwin)
    sq = _scale(*ae, bcnt)
    an = _seg_hop(sq, zrows, hwin, nwin)
    return _fin3(*an, dcnt, b3[None])


# speedup vs baseline: 11.1054x; 1.9249x over previous
import functools

import jax
import jax.numpy as jnp
from jax import lax
from jax.experimental import pallas as pl
from jax.experimental.pallas import tpu as pltpu
from jax.experimental.pallas import tpu_sc as plsc

N = 10000
E = 320000
F = 64
EPS = 1e-5

NC, NS, WLEN = 2, 16, 128
NWIN = 160
KG = 2
NG = NWIN // KG
EPAD = NS * NWIN * WLEN
NPAD = 10112
RP = NPAD // NS

_MESH = plsc.VectorSubcoreMesh(
    core_axis_name="c", subcore_axis_name="s", num_cores=NC, num_subcores=NS)


@functools.partial(
    pl.kernel,
    out_type=[jax.ShapeDtypeStruct((NPAD, F), jnp.float32)] * 2,
    mesh=_MESH,
    compiler_params=pltpu.CompilerParams(use_tc_tiling_on_sc=False),
    scratch_types=[
        pltpu.VMEM((NWIN, WLEN), jnp.int32),
        pltpu.VMEM((NWIN, WLEN), jnp.int32),
        pltpu.VMEM((2 * KG, WLEN, F), jnp.float32),
        pltpu.VMEM_SHARED((NPAD, F), jnp.float32),
        pltpu.SemaphoreType.DMA,
        pltpu.SemaphoreType.DMA,
    ],
)
def _hop(tab0, tab1, zrows, sidx_hbm, didx_hbm, out0, out1,
         sidx, didx, rows, acc, gsem, ssem):
    c = lax.axis_index("c")
    s = lax.axis_index("s")
    pltpu.sync_copy(sidx_hbm.at[s], sidx)
    pltpu.sync_copy(didx_hbm.at[s], didx)
    pltpu.sync_copy(zrows, acc.at[pl.ds(s * RP, RP)])
    plsc.subcore_barrier()

    def run(tab, out):
        def gfire(g, bank):
            for b in range(KG):
                pltpu.async_copy(
                    tab.at[sidx.at[g * KG + b]], rows.at[bank * KG + b], gsem)

        def gwait(g, bank):
            for b in range(KG):
                pltpu.make_async_copy(
                    tab.at[sidx.at[g * KG + b]], rows.at[bank * KG + b],
                    gsem).wait()

        def scat(g, bank):
            cps = [
                pltpu.async_copy(rows.at[bank * KG + b],
                                 acc.at[didx.at[g * KG + b]], ssem, add=True)
                for b in range(KG)
            ]
            for cp in cps:
                cp.wait()

        gfire(0, 0)

        def body(h, carry):
            g0 = 2 * h
            gfire(g0 + 1, 1)
            gwait(g0, 0)
            scat(g0, 0)

            @pl.when(h < NG // 2 - 1)
            def _():
                gfire(g0 + 2, 0)

            gwait(g0 + 1, 1)
            scat(g0 + 1, 1)
            return carry

        lax.fori_loop(0, NG // 2, body, 0)
        plsc.subcore_barrier()
        pltpu.sync_copy(acc.at[pl.ds(s * RP, RP)], out.at[pl.ds(s * RP, RP)])

    @pl.when(c == 0)
    def _():
        run(tab0, out0)

    @pl.when(c == 1)
    def _():
        run(tab1, out1)


@functools.partial(
    pl.kernel,
    out_type=[jax.ShapeDtypeStruct((NPAD, 16), jnp.float32)] * 2,
    mesh=_MESH,
    compiler_params=pltpu.CompilerParams(use_tc_tiling_on_sc=False),
    scratch_types=[
        pltpu.VMEM((NWIN, WLEN), jnp.int32),
        pltpu.VMEM((WLEN, 16), jnp.float32),
        pltpu.VMEM_SHARED((NPAD, 16), jnp.float32),
        pltpu.SemaphoreType.DMA,
    ],
)
def _counts(ones_hbm, z16, nidx_hbm, hidx_hbm, dcnt, bcnt, idx, ones, acc,
            ssem):
    c = lax.axis_index("c")
    s = lax.axis_index("s")
    pltpu.sync_copy(ones_hbm, ones)
    pltpu.sync_copy(z16, acc.at[pl.ds(s * RP, RP)])

    def run(idx_hbm, out):
        pltpu.sync_copy(idx_hbm.at[s], idx)
        plsc.subcore_barrier()

        def grp(g, carry):
            cps = [
                pltpu.async_copy(ones, acc.at[idx.at[g * KG + b]], ssem,
                                 add=True)
                for b in range(KG)
            ]
            for cp in cps:
                cp.wait()
            return carry
        lax.fori_loop(0, NG, grp, 0)
        plsc.subcore_barrier()
        pltpu.sync_copy(acc.at[pl.ds(s * RP, RP)], out.at[pl.ds(s * RP, RP)])

    @pl.when(c == 0)
    def _():
        run(nidx_hbm, dcnt)

    @pl.when(c == 1)
    def _():
        run(hidx_hbm, bcnt)


def _mm1_body(x_ref, w_ref, *o_refs):
    xw = jnp.dot(x_ref[...], w_ref[...], preferred_element_type=jnp.float32)
    for q, o in enumerate(o_refs):
        o[...] = xw[:, q * F:(q + 1) * F]


_mm1 = pl.pallas_call(
    _mm1_body,
    out_shape=[jax.ShapeDtypeStruct((NPAD, F), jnp.float32)] * 4,
)


def _scale_body(a0, a1, a2, a3, cnt_ref, o0, o1, o2, o3):
    cnt = cnt_ref[...][:, :1]
    inv = jnp.where(cnt > 0, 1.0 / cnt, 0.0)
    for a, o in ((a0, o0), (a1, o1), (a2, o2), (a3, o3)):
        o[...] = a[...] * inv


_scale = pl.pallas_call(
    _scale_body,
    out_shape=[jax.ShapeDtypeStruct((NPAD, F), jnp.float32)] * 4,
)


def _fin_body(a0, a1, a2, a3, cnt_ref, b_ref, g_ref, bt_ref, w_ref, *o_refs):
    t = jnp.concatenate([a0[...], a1[...], a2[...], a3[...]], axis=1)
    cnt = cnt_ref[...][:, :1]
    inv = jnp.where(cnt > 0, 1.0 / cnt, 0.0)
    t = jnp.maximum(t * inv + b_ref[...], 0.0)[:N]
    m = jnp.mean(t, axis=0, keepdims=True)
    v = jnp.mean(jnp.square(t - m), axis=0, keepdims=True)
    t = (t - m) * lax.rsqrt(v + EPS) * g_ref[...] + bt_ref[...]
    xw = jnp.dot(t, w_ref[...], preferred_element_type=jnp.float32)
    xw = jnp.concatenate([xw, jnp.zeros((NPAD - N, 4 * F), xw.dtype)], axis=0)
    for q, o in enumerate(o_refs):
        o[...] = xw[:, q * F:(q + 1) * F]


_fin = pl.pallas_call(
    _fin_body,
    out_shape=[jax.ShapeDtypeStruct((NPAD, F), jnp.float32)] * 4,
)


def _fin3_body(a0, a1, a2, a3, cnt_ref, b_ref, o_ref):
    t = jnp.concatenate([a0[...], a1[...], a2[...], a3[...]], axis=1)
    cnt = cnt_ref[...][:, :1]
    inv = jnp.where(cnt > 0, 1.0 / cnt, 0.0)
    o_ref[...] = jnp.maximum(t * inv + b_ref[...], 0.0)[:N]


_fin3 = pl.pallas_call(
    _fin3_body,
    out_shape=jax.ShapeDtypeStruct((N, 4 * F), jnp.float32),
)


def _tile_windows(idx):
    pad = jnp.arange(EPAD - E, dtype=jnp.int32) % (NPAD - N) + N
    return jnp.concatenate([idx.astype(jnp.int32), pad]).reshape(NS, NWIN, WLEN)


def _seg_hop(q, zrows, sidx, didx):
    a0, a1 = _hop(q[0], q[1], zrows, sidx, didx)
    a2, a3 = _hop(q[2], q[3], zrows, sidx, didx)
    return (a0, a1, a2, a3)


def kernel(x, edge, W1, b1, g1, bt1, W2, b2, g2, bt2, W3, b3):
    nwin = _tile_windows(edge[0])
    hwin = _tile_windows(edge[1])
    zrows = jnp.zeros((RP, F), jnp.float32)
    z16 = jnp.zeros((RP, 16), jnp.float32)
    ones = jnp.ones((WLEN, 16), jnp.float32)

    dcnt, bcnt = _counts(ones, z16, nwin, hwin)
    xp = jnp.pad(x, ((0, NPAD - N), (0, 0)))
    q = _mm1(xp, W1)

    for b, g, bt, wn in ((b1, g1, bt1, W2), (b2, g2, bt2, W3)):
        ae = _seg_hop(q, zrows, nwin, hwin)
        sq = _scale(*ae, bcnt)
        an = _seg_hop(sq, zrows, hwin, nwin)
        q = _fin(*an, dcnt, b[None], g[None], bt[None], wn)

    ae = _seg_hop(q, zrows, nwin, hwin)
    sq = _scale(*ae, bcnt)
    an = _seg_hop(sq, zrows, hwin, nwin)
    return _fin3(*an, dcnt, b3[None])


# speedup vs baseline: 11.2737x; 1.0152x over previous
import functools

import jax
import jax.numpy as jnp
from jax import lax
from jax.experimental import pallas as pl
from jax.experimental.pallas import tpu as pltpu
from jax.experimental.pallas import tpu_sc as plsc

N = 10000
E = 320000
F = 64
EPS = 1e-5

NC, NS, WLEN = 2, 16, 128
NWIN = 160
KG = 2
NG = NWIN // KG
EPAD = NS * NWIN * WLEN
NPAD = 10112
RP = NPAD // NS

_MESH = plsc.VectorSubcoreMesh(
    core_axis_name="c", subcore_axis_name="s", num_cores=NC, num_subcores=NS)


@functools.partial(
    pl.kernel,
    out_type=[jax.ShapeDtypeStruct((NPAD, F), jnp.float32)] * 4,
    mesh=_MESH,
    compiler_params=pltpu.CompilerParams(use_tc_tiling_on_sc=False),
    scratch_types=[
        pltpu.VMEM((NWIN, WLEN), jnp.int32),
        pltpu.VMEM((NWIN, WLEN), jnp.int32),
        pltpu.VMEM((2 * KG, WLEN, F), jnp.float32),
        pltpu.VMEM_SHARED((NPAD, F), jnp.float32),
        pltpu.SemaphoreType.DMA,
        pltpu.SemaphoreType.DMA,
    ],
)
def _hop(tab0, tab1, tab2, tab3, zrows, sidx_hbm, didx_hbm,
         out0, out1, out2, out3, sidx, didx, rows, acc, gsem, ssem):
    c = lax.axis_index("c")
    s = lax.axis_index("s")
    pltpu.sync_copy(sidx_hbm.at[s], sidx)
    pltpu.sync_copy(didx_hbm.at[s], didx)
    pltpu.sync_copy(zrows, acc.at[pl.ds(s * RP, RP)])
    plsc.subcore_barrier()

    def run(tab, out):
        def gfire(g, bank):
            for b in range(KG):
                pltpu.async_copy(
                    tab.at[sidx.at[g * KG + b]], rows.at[bank * KG + b], gsem)

        def gwait(g, bank):
            for b in range(KG):
                pltpu.make_async_copy(
                    tab.at[sidx.at[g * KG + b]], rows.at[bank * KG + b],
                    gsem).wait()

        def scat(g, bank):
            cps = [
                pltpu.async_copy(rows.at[bank * KG + b],
                                 acc.at[didx.at[g * KG + b]], ssem, add=True)
                for b in range(KG)
            ]
            for cp in cps:
                cp.wait()

        gfire(0, 0)

        def body(h, carry):
            g0 = 2 * h
            gfire(g0 + 1, 1)
            gwait(g0, 0)
            scat(g0, 0)

            @pl.when(h < NG // 2 - 1)
            def _():
                gfire(g0 + 2, 0)

            gwait(g0 + 1, 1)
            scat(g0 + 1, 1)
            return carry

        lax.fori_loop(0, NG // 2, body, 0)
        plsc.subcore_barrier()
        pltpu.sync_copy(acc.at[pl.ds(s * RP, RP)], out.at[pl.ds(s * RP, RP)])

    @pl.when(c == 0)
    def _():
        run(tab0, out0)

    @pl.when(c == 1)
    def _():
        run(tab1, out1)

    pltpu.sync_copy(zrows, acc.at[pl.ds(s * RP, RP)])
    plsc.subcore_barrier()

    @pl.when(c == 0)
    def _():
        run(tab2, out2)

    @pl.when(c == 1)
    def _():
        run(tab3, out3)


@functools.partial(
    pl.kernel,
    out_type=[jax.ShapeDtypeStruct((NPAD, 16), jnp.float32)] * 2,
    mesh=_MESH,
    compiler_params=pltpu.CompilerParams(use_tc_tiling_on_sc=False),
    scratch_types=[
        pltpu.VMEM((NWIN, WLEN), jnp.int32),
        pltpu.VMEM((WLEN, 16), jnp.float32),
        pltpu.VMEM_SHARED((NPAD, 16), jnp.float32),
        pltpu.SemaphoreType.DMA,
    ],
)
def _counts(ones_hbm, z16, nidx_hbm, hidx_hbm, dcnt, bcnt, idx, ones, acc,
            ssem):
    c = lax.axis_index("c")
    s = lax.axis_index("s")
    pltpu.sync_copy(ones_hbm, ones)
    pltpu.sync_copy(z16, acc.at[pl.ds(s * RP, RP)])

    def run(idx_hbm, out):
        pltpu.sync_copy(idx_hbm.at[s], idx)
        plsc.subcore_barrier()

        def grp(g, carry):
            cps = [
                pltpu.async_copy(ones, acc.at[idx.at[g * KG + b]], ssem,
                                 add=True)
                for b in range(KG)
            ]
            for cp in cps:
                cp.wait()
            return carry
        lax.fori_loop(0, NG, grp, 0)
        plsc.subcore_barrier()
        pltpu.sync_copy(acc.at[pl.ds(s * RP, RP)], out.at[pl.ds(s * RP, RP)])

    @pl.when(c == 0)
    def _():
        run(nidx_hbm, dcnt)

    @pl.when(c == 1)
    def _():
        run(hidx_hbm, bcnt)


def _mm1_body(x_ref, w_ref, *o_refs):
    xw = jnp.dot(x_ref[...], w_ref[...], preferred_element_type=jnp.float32)
    for q, o in enumerate(o_refs):
        o[...] = xw[:, q * F:(q + 1) * F]


_mm1 = pl.pallas_call(
    _mm1_body,
    out_shape=[jax.ShapeDtypeStruct((NPAD, F), jnp.float32)] * 4,
)


def _scale_body(a0, a1, a2, a3, cnt_ref, o0, o1, o2, o3):
    cnt = cnt_ref[...][:, :1]
    inv = jnp.where(cnt > 0, 1.0 / cnt, 0.0)
    for a, o in ((a0, o0), (a1, o1), (a2, o2), (a3, o3)):
        o[...] = a[...] * inv


_scale = pl.pallas_call(
    _scale_body,
    out_shape=[jax.ShapeDtypeStruct((NPAD, F), jnp.float32)] * 4,
)


def _fin_body(a0, a1, a2, a3, cnt_ref, b_ref, g_ref, bt_ref, w_ref, *o_refs):
    t = jnp.concatenate([a0[...], a1[...], a2[...], a3[...]], axis=1)
    cnt = cnt_ref[...][:, :1]
    inv = jnp.where(cnt > 0, 1.0 / cnt, 0.0)
    t = jnp.maximum(t * inv + b_ref[...], 0.0)[:N]
    m = jnp.mean(t, axis=0, keepdims=True)
    v = jnp.mean(jnp.square(t - m), axis=0, keepdims=True)
    t = (t - m) * lax.rsqrt(v + EPS) * g_ref[...] + bt_ref[...]
    xw = jnp.dot(t, w_ref[...], preferred_element_type=jnp.float32)
    xw = jnp.concatenate([xw, jnp.zeros((NPAD - N, 4 * F), xw.dtype)], axis=0)
    for q, o in enumerate(o_refs):
        o[...] = xw[:, q * F:(q + 1) * F]


_fin = pl.pallas_call(
    _fin_body,
    out_shape=[jax.ShapeDtypeStruct((NPAD, F), jnp.float32)] * 4,
)


def _fin3_body(a0, a1, a2, a3, cnt_ref, b_ref, o_ref):
    t = jnp.concatenate([a0[...], a1[...], a2[...], a3[...]], axis=1)
    cnt = cnt_ref[...][:, :1]
    inv = jnp.where(cnt > 0, 1.0 / cnt, 0.0)
    o_ref[...] = jnp.maximum(t * inv + b_ref[...], 0.0)[:N]


_fin3 = pl.pallas_call(
    _fin3_body,
    out_shape=jax.ShapeDtypeStruct((N, 4 * F), jnp.float32),
)


def _tile_windows(idx):
    pad = jnp.arange(EPAD - E, dtype=jnp.int32) % (NPAD - N) + N
    return jnp.concatenate([idx.astype(jnp.int32), pad]).reshape(NS, NWIN, WLEN)


def _seg_hop(q, zrows, sidx, didx):
    return _hop(q[0], q[1], q[2], q[3], zrows, sidx, didx)


def kernel(x, edge, W1, b1, g1, bt1, W2, b2, g2, bt2, W3, b3):
    nwin = _tile_windows(edge[0])
    hwin = _tile_windows(edge[1])
    zrows = jnp.zeros((RP, F), jnp.float32)
    z16 = jnp.zeros((RP, 16), jnp.float32)
    ones = jnp.ones((WLEN, 16), jnp.float32)

    dcnt, bcnt = _counts(ones, z16, nwin, hwin)
    xp = jnp.pad(x, ((0, NPAD - N), (0, 0)))
    q = _mm1(xp, W1)

    for b, g, bt, wn in ((b1, g1, bt1, W2), (b2, g2, bt2, W3)):
        ae = _seg_hop(q, zrows, nwin, hwin)
        sq = _scale(*ae, bcnt)
        an = _seg_hop(sq, zrows, hwin, nwin)
        q = _fin(*an, dcnt, b[None], g[None], bt[None], wn)

    ae = _seg_hop(q, zrows, nwin, hwin)
    sq = _scale(*ae, bcnt)
    an = _seg_hop(sq, zrows, hwin, nwin)
    return _fin3(*an, dcnt, b3[None])


# speedup vs baseline: 11.2767x; 1.0003x over previous
import functools

import jax
import jax.numpy as jnp
from jax import lax
from jax.experimental import pallas as pl
from jax.experimental.pallas import tpu as pltpu
from jax.experimental.pallas import tpu_sc as plsc

N = 10000
E = 320000
F = 64
EPS = 1e-5

NC, NS, WLEN = 2, 16, 256
NWIN = 80
KG = 1
NG = NWIN // KG
EPAD = NS * NWIN * WLEN
NPAD = 10112
RP = NPAD // NS

_MESH = plsc.VectorSubcoreMesh(
    core_axis_name="c", subcore_axis_name="s", num_cores=NC, num_subcores=NS)


@functools.partial(
    pl.kernel,
    out_type=[jax.ShapeDtypeStruct((NPAD, F), jnp.float32)] * 4,
    mesh=_MESH,
    compiler_params=pltpu.CompilerParams(use_tc_tiling_on_sc=False),
    scratch_types=[
        pltpu.VMEM((NWIN, WLEN), jnp.int32),
        pltpu.VMEM((NWIN, WLEN), jnp.int32),
        pltpu.VMEM((2 * KG, WLEN, F), jnp.float32),
        pltpu.VMEM_SHARED((NPAD, F), jnp.float32),
        pltpu.SemaphoreType.DMA,
        pltpu.SemaphoreType.DMA,
    ],
)
def _hop(tab0, tab1, tab2, tab3, zrows, sidx_hbm, didx_hbm,
         out0, out1, out2, out3, sidx, didx, rows, acc, gsem, ssem):
    c = lax.axis_index("c")
    s = lax.axis_index("s")
    pltpu.sync_copy(sidx_hbm.at[s], sidx)
    pltpu.sync_copy(didx_hbm.at[s], didx)
    pltpu.sync_copy(zrows, acc.at[pl.ds(s * RP, RP)])
    plsc.subcore_barrier()

    def run(tab, out):
        def gfire(g, bank):
            for b in range(KG):
                pltpu.async_copy(
                    tab.at[sidx.at[g * KG + b]], rows.at[bank * KG + b], gsem)

        def gwait(g, bank):
            for b in range(KG):
                pltpu.make_async_copy(
                    tab.at[sidx.at[g * KG + b]], rows.at[bank * KG + b],
                    gsem).wait()

        def scat(g, bank):
            cps = [
                pltpu.async_copy(rows.at[bank * KG + b],
                                 acc.at[didx.at[g * KG + b]], ssem, add=True)
                for b in range(KG)
            ]
            for cp in cps:
                cp.wait()

        gfire(0, 0)

        def body(h, carry):
            g0 = 2 * h
            gfire(g0 + 1, 1)
            gwait(g0, 0)
            scat(g0, 0)

            @pl.when(h < NG // 2 - 1)
            def _():
                gfire(g0 + 2, 0)

            gwait(g0 + 1, 1)
            scat(g0 + 1, 1)
            return carry

        lax.fori_loop(0, NG // 2, body, 0)
        plsc.subcore_barrier()
        pltpu.sync_copy(acc.at[pl.ds(s * RP, RP)], out.at[pl.ds(s * RP, RP)])

    @pl.when(c == 0)
    def _():
        run(tab0, out0)

    @pl.when(c == 1)
    def _():
        run(tab1, out1)

    pltpu.sync_copy(zrows, acc.at[pl.ds(s * RP, RP)])
    plsc.subcore_barrier()

    @pl.when(c == 0)
    def _():
        run(tab2, out2)

    @pl.when(c == 1)
    def _():
        run(tab3, out3)


@functools.partial(
    pl.kernel,
    out_type=[jax.ShapeDtypeStruct((NPAD, 16), jnp.float32)] * 2,
    mesh=_MESH,
    compiler_params=pltpu.CompilerParams(use_tc_tiling_on_sc=False),
    scratch_types=[
        pltpu.VMEM((NWIN, WLEN), jnp.int32),
        pltpu.VMEM((WLEN, 16), jnp.float32),
        pltpu.VMEM_SHARED((NPAD, 16), jnp.float32),
        pltpu.SemaphoreType.DMA,
    ],
)
def _counts(ones_hbm, z16, nidx_hbm, hidx_hbm, dcnt, bcnt, idx, ones, acc,
            ssem):
    c = lax.axis_index("c")
    s = lax.axis_index("s")
    pltpu.sync_copy(ones_hbm, ones)
    pltpu.sync_copy(z16, acc.at[pl.ds(s * RP, RP)])

    def run(idx_hbm, out):
        pltpu.sync_copy(idx_hbm.at[s], idx)
        plsc.subcore_barrier()

        def grp(g, carry):
            cps = [
                pltpu.async_copy(ones, acc.at[idx.at[g * KG + b]], ssem,
                                 add=True)
                for b in range(KG)
            ]
            for cp in cps:
                cp.wait()
            return carry
        lax.fori_loop(0, NG, grp, 0)
        plsc.subcore_barrier()
        pltpu.sync_copy(acc.at[pl.ds(s * RP, RP)], out.at[pl.ds(s * RP, RP)])

    @pl.when(c == 0)
    def _():
        run(nidx_hbm, dcnt)

    @pl.when(c == 1)
    def _():
        run(hidx_hbm, bcnt)


def _mm1_body(x_ref, w_ref, *o_refs):
    xw = jnp.dot(x_ref[...], w_ref[...], preferred_element_type=jnp.float32)
    for q, o in enumerate(o_refs):
        o[...] = xw[:, q * F:(q + 1) * F]


_mm1 = pl.pallas_call(
    _mm1_body,
    out_shape=[jax.ShapeDtypeStruct((NPAD, F), jnp.float32)] * 4,
)


def _scale_body(a0, a1, a2, a3, cnt_ref, o0, o1, o2, o3):
    cnt = cnt_ref[...][:, :1]
    inv = jnp.where(cnt > 0, 1.0 / cnt, 0.0)
    for a, o in ((a0, o0), (a1, o1), (a2, o2), (a3, o3)):
        o[...] = a[...] * inv


_scale = pl.pallas_call(
    _scale_body,
    out_shape=[jax.ShapeDtypeStruct((NPAD, F), jnp.float32)] * 4,
)


def _fin_body(a0, a1, a2, a3, cnt_ref, b_ref, g_ref, bt_ref, w_ref, *o_refs):
    t = jnp.concatenate([a0[...], a1[...], a2[...], a3[...]], axis=1)
    cnt = cnt_ref[...][:, :1]
    inv = jnp.where(cnt > 0, 1.0 / cnt, 0.0)
    t = jnp.maximum(t * inv + b_ref[...], 0.0)[:N]
    m = jnp.mean(t, axis=0, keepdims=True)
    v = jnp.mean(jnp.square(t - m), axis=0, keepdims=True)
    t = (t - m) * lax.rsqrt(v + EPS) * g_ref[...] + bt_ref[...]
    xw = jnp.dot(t, w_ref[...], preferred_element_type=jnp.float32)
    xw = jnp.concatenate([xw, jnp.zeros((NPAD - N, 4 * F), xw.dtype)], axis=0)
    for q, o in enumerate(o_refs):
        o[...] = xw[:, q * F:(q + 1) * F]


_fin = pl.pallas_call(
    _fin_body,
    out_shape=[jax.ShapeDtypeStruct((NPAD, F), jnp.float32)] * 4,
)


def _fin3_body(a0, a1, a2, a3, cnt_ref, b_ref, o_ref):
    t = jnp.concatenate([a0[...], a1[...], a2[...], a3[...]], axis=1)
    cnt = cnt_ref[...][:, :1]
    inv = jnp.where(cnt > 0, 1.0 / cnt, 0.0)
    o_ref[...] = jnp.maximum(t * inv + b_ref[...], 0.0)[:N]


_fin3 = pl.pallas_call(
    _fin3_body,
    out_shape=jax.ShapeDtypeStruct((N, 4 * F), jnp.float32),
)


def _tile_windows(idx):
    pad = jnp.arange(EPAD - E, dtype=jnp.int32) % (NPAD - N) + N
    return jnp.concatenate([idx.astype(jnp.int32), pad]).reshape(NS, NWIN, WLEN)


def _seg_hop(q, zrows, sidx, didx):
    return _hop(q[0], q[1], q[2], q[3], zrows, sidx, didx)


def kernel(x, edge, W1, b1, g1, bt1, W2, b2, g2, bt2, W3, b3):
    nwin = _tile_windows(edge[0])
    hwin = _tile_windows(edge[1])
    zrows = jnp.zeros((RP, F), jnp.float32)
    z16 = jnp.zeros((RP, 16), jnp.float32)
    ones = jnp.ones((WLEN, 16), jnp.float32)

    dcnt, bcnt = _counts(ones, z16, nwin, hwin)
    xp = jnp.pad(x, ((0, NPAD - N), (0, 0)))
    q = _mm1(xp, W1)

    for b, g, bt, wn in ((b1, g1, bt1, W2), (b2, g2, bt2, W3)):
        ae = _seg_hop(q, zrows, nwin, hwin)
        sq = _scale(*ae, bcnt)
        an = _seg_hop(sq, zrows, hwin, nwin)
        q = _fin(*an, dcnt, b[None], g[None], bt[None], wn)

    ae = _seg_hop(q, zrows, nwin, hwin)
    sq = _scale(*ae, bcnt)
    an = _seg_hop(sq, zrows, hwin, nwin)
    return _fin3(*an, dcnt, b3[None])


# speedup vs baseline: 12.3263x; 1.0931x over previous
import functools

import jax
import jax.numpy as jnp
from jax import lax
from jax.experimental import pallas as pl
from jax.experimental.pallas import tpu as pltpu
from jax.experimental.pallas import tpu_sc as plsc

N = 10000
E = 320000
F = 64
EPS = 1e-5

NC, NS, WLEN = 2, 16, 128
NWIN = 160
KG = 2
NG = NWIN // KG
EPAD = NS * NWIN * WLEN
NPAD = 10112
RP = NPAD // NS
CH = 128
NCH = 5

_MESH = plsc.VectorSubcoreMesh(
    core_axis_name="c", subcore_axis_name="s", num_cores=NC, num_subcores=NS)


def _stream_hop(tab, sidx, didx, rows, acc, gsem, ssem):
    def gfire(g, bank):
        for b in range(KG):
            pltpu.async_copy(
                tab.at[sidx.at[g * KG + b]], rows.at[bank * KG + b], gsem)

    def gwait(g, bank):
        for b in range(KG):
            pltpu.make_async_copy(
                tab.at[sidx.at[g * KG + b]], rows.at[bank * KG + b],
                gsem).wait()

    def scat(g, bank):
        cps = [
            pltpu.async_copy(rows.at[bank * KG + b],
                             acc.at[didx.at[g * KG + b]], ssem, add=True)
            for b in range(KG)
        ]
        for cp in cps:
            cp.wait()

    gfire(0, 0)

    def body(h, carry):
        g0 = 2 * h
        gfire(g0 + 1, 1)
        gwait(g0, 0)
        scat(g0, 0)

        @pl.when(h < NG // 2 - 1)
        def _():
            gfire(g0 + 2, 0)

        gwait(g0 + 1, 1)
        scat(g0 + 1, 1)
        return carry

    lax.fori_loop(0, NG // 2, body, 0)


def _scaled_copyout(s, acc, inv_t, buf, out):
    for ci in range(NCH):
        nc = CH if ci < NCH - 1 else RP - (NCH - 1) * CH
        base = s * RP + ci * CH
        pltpu.sync_copy(acc.at[pl.ds(base, nc)], buf.at[pl.ds(0, nc)])

        def srow(r, carry):
            inv = inv_t[ci * CH + r, :]
            for k in range(F // 16):
                sl = pl.ds(k * 16, 16)
                buf[r, sl] = buf[r, sl] * inv
            return carry

        lax.fori_loop(0, nc, srow, 0)
        pltpu.sync_copy(buf.at[pl.ds(0, nc)], out.at[pl.ds(base, nc)])


@functools.partial(
    pl.kernel,
    out_type=[jax.ShapeDtypeStruct((NPAD, F), jnp.float32)] * 8,
    mesh=_MESH,
    compiler_params=pltpu.CompilerParams(use_tc_tiling_on_sc=False),
    scratch_types=[
        pltpu.VMEM((NWIN, WLEN), jnp.int32),
        pltpu.VMEM((NWIN, WLEN), jnp.int32),
        pltpu.VMEM((2 * KG, WLEN, F), jnp.float32),
        pltpu.VMEM((RP, 16), jnp.float32),
        pltpu.VMEM_SHARED((NPAD, F), jnp.float32),
        pltpu.SemaphoreType.DMA,
        pltpu.SemaphoreType.DMA,
    ],
)
def _layer(t0, t1, t2, t3, binv_hbm, zrows, nidx_hbm, hidx_hbm,
           o0, o1, o2, o3, e0, e1, e2, e3,
           nidx, hidx, rows, binv_t, acc, gsem, ssem):
    c = lax.axis_index("c")
    s = lax.axis_index("s")
    pltpu.sync_copy(nidx_hbm.at[s], nidx)
    pltpu.sync_copy(hidx_hbm.at[s], hidx)
    pltpu.sync_copy(binv_hbm.at[pl.ds(s * RP, RP)], binv_t)

    def one_pass(tab, eq, out):
        pltpu.sync_copy(zrows, acc.at[pl.ds(s * RP, RP)])
        plsc.subcore_barrier()
        _stream_hop(tab, nidx, hidx, rows, acc, gsem, ssem)
        plsc.subcore_barrier()
        _scaled_copyout(s, acc, binv_t, rows.at[0], eq)
        pltpu.sync_copy(zrows, acc.at[pl.ds(s * RP, RP)])
        plsc.subcore_barrier()
        _stream_hop(eq, hidx, nidx, rows, acc, gsem, ssem)
        plsc.subcore_barrier()
        pltpu.sync_copy(acc.at[pl.ds(s * RP, RP)], out.at[pl.ds(s * RP, RP)])

    @pl.when(c == 0)
    def _():
        one_pass(t0, e0, o0)

    @pl.when(c == 1)
    def _():
        one_pass(t1, e1, o1)

    plsc.subcore_barrier()

    @pl.when(c == 0)
    def _():
        one_pass(t2, e2, o2)

    @pl.when(c == 1)
    def _():
        one_pass(t3, e3, o3)


@functools.partial(
    pl.kernel,
    out_type=[jax.ShapeDtypeStruct((NPAD, 16), jnp.float32)] * 2,
    mesh=_MESH,
    compiler_params=pltpu.CompilerParams(use_tc_tiling_on_sc=False),
    scratch_types=[
        pltpu.VMEM((NWIN, WLEN), jnp.int32),
        pltpu.VMEM((WLEN, 16), jnp.float32),
        pltpu.VMEM((CH, 16), jnp.float32),
        pltpu.VMEM_SHARED((NPAD, 16), jnp.float32),
        pltpu.SemaphoreType.DMA,
    ],
)
def _counts(ones_hbm, z16, nidx_hbm, hidx_hbm, dinv, binv, idx, ones, buf,
            acc, ssem):
    c = lax.axis_index("c")
    s = lax.axis_index("s")
    pltpu.sync_copy(ones_hbm, ones)
    pltpu.sync_copy(z16, acc.at[pl.ds(s * RP, RP)])

    def run(idx_hbm, out):
        pltpu.sync_copy(idx_hbm.at[s], idx)
        plsc.subcore_barrier()

        def grp(g, carry):
            cps = [
                pltpu.async_copy(ones, acc.at[idx.at[g * KG + b]], ssem,
                                 add=True)
                for b in range(KG)
            ]
            for cp in cps:
                cp.wait()
            return carry
        lax.fori_loop(0, NG, grp, 0)
        plsc.subcore_barrier()

        for ci in range(NCH):
            nc = CH if ci < NCH - 1 else RP - (NCH - 1) * CH
            base = s * RP + ci * CH
            pltpu.sync_copy(acc.at[pl.ds(base, nc)], buf.at[pl.ds(0, nc)])

            def rrow(r, carry):
                cnt = buf[r, :]
                buf[r, :] = jnp.where(cnt > 0.0, 1.0 / cnt, 0.0)
                return carry

            lax.fori_loop(0, nc, rrow, 0)
            pltpu.sync_copy(buf.at[pl.ds(0, nc)], out.at[pl.ds(base, nc)])

    @pl.when(c == 0)
    def _():
        run(nidx_hbm, dinv)

    @pl.when(c == 1)
    def _():
        run(hidx_hbm, binv)


def _mm1_body(x_ref, w_ref, *o_refs):
    xw = jnp.dot(x_ref[...], w_ref[...], preferred_element_type=jnp.float32)
    for q, o in enumerate(o_refs):
        o[...] = xw[:, q * F:(q + 1) * F]


_mm1 = pl.pallas_call(
    _mm1_body,
    out_shape=[jax.ShapeDtypeStruct((NPAD, F), jnp.float32)] * 4,
)


def _fin_body(a0, a1, a2, a3, dinv_ref, b_ref, g_ref, bt_ref, w_ref, *o_refs):
    t = jnp.concatenate([a0[...], a1[...], a2[...], a3[...]], axis=1)
    t = jnp.maximum(t * dinv_ref[...][:, :1] + b_ref[...], 0.0)[:N]
    m = jnp.mean(t, axis=0, keepdims=True)
    v = jnp.mean(jnp.square(t - m), axis=0, keepdims=True)
    t = (t - m) * lax.rsqrt(v + EPS) * g_ref[...] + bt_ref[...]
    xw = jnp.dot(t, w_ref[...], preferred_element_type=jnp.float32)
    xw = jnp.concatenate([xw, jnp.zeros((NPAD - N, 4 * F), xw.dtype)], axis=0)
    for q, o in enumerate(o_refs):
        o[...] = xw[:, q * F:(q + 1) * F]


_fin = pl.pallas_call(
    _fin_body,
    out_shape=[jax.ShapeDtypeStruct((NPAD, F), jnp.float32)] * 4,
)


def _fin3_body(a0, a1, a2, a3, dinv_ref, b_ref, o_ref):
    t = jnp.concatenate([a0[...], a1[...], a2[...], a3[...]], axis=1)
    o_ref[...] = jnp.maximum(t * dinv_ref[...][:, :1] + b_ref[...], 0.0)[:N]


_fin3 = pl.pallas_call(
    _fin3_body,
    out_shape=jax.ShapeDtypeStruct((N, 4 * F), jnp.float32),
)


def _tile_windows(idx):
    pad = jnp.arange(EPAD - E, dtype=jnp.int32) % (NPAD - N) + N
    return jnp.concatenate([idx.astype(jnp.int32), pad]).reshape(NS, NWIN, WLEN)


def kernel(x, edge, W1, b1, g1, bt1, W2, b2, g2, bt2, W3, b3):
    nwin = _tile_windows(edge[0])
    hwin = _tile_windows(edge[1])
    zrows = jnp.zeros((RP, F), jnp.float32)
    z16 = jnp.zeros((RP, 16), jnp.float32)
    ones = jnp.ones((WLEN, 16), jnp.float32)

    dinv, binv = _counts(ones, z16, nwin, hwin)
    xp = jnp.pad(x, ((0, NPAD - N), (0, 0)))
    q = _mm1(xp, W1)

    for b, g, bt, wn in ((b1, g1, bt1, W2), (b2, g2, bt2, W3)):
        an = _layer(*q, binv, zrows, nwin, hwin)[:4]
        q = _fin(*an, dinv, b[None], g[None], bt[None], wn)

    an = _layer(*q, binv, zrows, nwin, hwin)[:4]
    return _fin3(*an, dinv, b3[None])
